# baseline (speedup 1.0000x reference)
import functools

import jax
import jax.numpy as jnp
from jax.experimental import pallas as pl
from jax.experimental.pallas import tpu as pltpu


def _matmul_body(x_ref, w_ref, b_ref, o_ref):
    o_ref[...] = (
        jnp.dot(x_ref[...], w_ref[...], preferred_element_type=jnp.float32)
        + b_ref[...]
    )


def kernel(in_core_feats, aux_feats, id_map, roi_ids, pos_ids, weight, bias):
    kH, kW = 3, 3
    N, C = in_core_feats.shape
    A = aux_feats.shape[0]
    O = weight.shape[0]
    rH, rW = id_map.shape[1], id_map.shape[2]

    offs_y = jnp.repeat(jnp.arange(-1, 2), kW)
    offs_x = jnp.tile(jnp.arange(-1, 2), kH)
    pos_y = pos_ids[:, 1, None] + offs_y[None, :]
    pos_x = pos_ids[:, 0, None] + offs_x[None, :]
    pad_mask = (pos_y < 0) | (pos_y >= rH) | (pos_x < 0) | (pos_x >= rW)
    roi_b = jnp.broadcast_to(roi_ids[:, None], pos_y.shape)
    pos_y = jnp.clip(pos_y, 0, rH - 1)
    pos_x = jnp.clip(pos_x, 0, rW - 1)
    conv_ids = id_map[roi_b, pos_y, pos_x]
    pad_idx = N + A
    conv_ids = jnp.where(pad_mask, pad_idx, conv_ids)
    zero_row = jnp.zeros((1, C), dtype=in_core_feats.dtype)
    all_feats = jnp.concatenate([in_core_feats, aux_feats, zero_row], axis=0)
    flat = jnp.take(all_feats, conv_ids, axis=0).reshape(N, kH * kW * C)

    BN = 512
    n_pad = (N + BN - 1) // BN * BN
    flat = jnp.pad(flat, ((0, n_pad - N), (0, 0)))
    K = kH * kW * C

    out = pl.pallas_call(
        _matmul_body,
        grid=(n_pad // BN,),
        in_specs=[
            pl.BlockSpec((BN, K), lambda i: (i, 0)),
            pl.BlockSpec((K, O), lambda i: (0, 0)),
            pl.BlockSpec((1, O), lambda i: (0, 0)),
        ],
        out_specs=pl.BlockSpec((BN, O), lambda i: (i, 0)),
        out_shape=jax.ShapeDtypeStruct((n_pad, O), jnp.float32),
    )(flat, weight.T, bias[None, :])
    return out[:N]


# speedup vs baseline: 3.6306x; 3.6306x over previous
import functools

import jax
import jax.numpy as jnp
from jax import lax
from jax.experimental import pallas as pl
from jax.experimental.pallas import tpu as pltpu
from jax.experimental.pallas import tpu_sc as plsc

L = 16
NC, NS = 2, 16
NW = NC * NS
CN = 32
TAPS = 9


def _matmul_body(x_ref, w_ref, o_ref):
    res = jnp.dot(x_ref[...], w_ref[...], preferred_element_type=jnp.float32)
    for k in range(TAPS):
        o_ref[k, ...] = res[:, k * 128:(k + 1) * 128]


def _compute_y(all_feats, wbig):
    Vp, C = all_feats.shape
    BM = 512
    return pl.pallas_call(
        _matmul_body,
        grid=(Vp // BM,),
        in_specs=[
            pl.BlockSpec((BM, C), lambda i: (i, 0)),
            pl.BlockSpec((C, TAPS * C), lambda i: (0, 0)),
        ],
        out_specs=pl.BlockSpec((TAPS, BM, C), lambda i: (0, i, 0)),
        out_shape=jax.ShapeDtypeStruct((TAPS, Vp, C), jnp.float32),
    )(all_feats, wbig)


def _make_sc_gather(Np, rW, pad_v):
    per_w = Np // NW
    chunks = per_w // CN
    mesh = plsc.VectorSubcoreMesh(core_axis_name="c", subcore_axis_name="s")

    @functools.partial(
        pl.kernel,
        out_type=jax.ShapeDtypeStruct((Np, 128), jnp.float32),
        mesh=mesh,
        compiler_params=pltpu.CompilerParams(needs_layout_passes=False),
        scratch_types=[
            pltpu.VMEM((CN,), jnp.int32),
            pltpu.VMEM((CN,), jnp.int32),
            pltpu.VMEM((CN,), jnp.int32),
            pltpu.VMEM((CN,), jnp.int32),
            pltpu.VMEM((2 * CN,), jnp.int32),
            pltpu.VMEM((2 * CN, 2 * rW), jnp.int32),
            pltpu.VMEM((TAPS, CN), jnp.int32),
            pltpu.VMEM((TAPS * CN, 128), jnp.float32),
            pltpu.VMEM((128,), jnp.float32),
            pltpu.VMEM((CN, 128), jnp.float32),
            pltpu.SemaphoreType.DMA,
        ],
    )
    def sc_kernel(idmap_hbm, roi_hbm, posx_hbm, posy_hbm, y_hbm, bias_hbm,
                  out_hbm, roi_v, posx_v, posy_v, p0_v, rowidx_v, rows_v,
                  yidx_v, ybuf_v, bias_v, outbuf_v, sem):
        wid = lax.axis_index("s") * NC + lax.axis_index("c")
        w_base = wid * per_w
        pltpu.sync_copy(bias_hbm, bias_v)

        def chunk_body(g, carry):
            base = w_base + g * CN
            pltpu.sync_copy(roi_hbm.at[pl.ds(base, CN)], roi_v)
            pltpu.sync_copy(posx_hbm.at[pl.ds(base, CN)], posx_v)
            pltpu.sync_copy(posy_hbm.at[pl.ds(base, CN)], posy_v)

            n_pairs = rW // 2
            for t in range(CN // L):
                roi = roi_v[pl.ds(t * L, L)]
                py = posy_v[pl.ds(t * L, L)]
                ylo = jnp.clip(py - 1, 0, rW - 1)
                p0 = jnp.minimum(ylo >> 1, n_pairs - 2)
                p0_v[pl.ds(t * L, L)] = p0
                base_pair = roi * n_pairs + p0
                rowidx_v[pl.ds(t * L, L)] = base_pair
                rowidx_v[pl.ds(CN + t * L, L)] = base_pair + 1
            pltpu.async_copy(idmap_hbm.at[rowidx_v], rows_v, sem).wait()

            for t in range(CN // L):
                px = posx_v[pl.ds(t * L, L)]
                py = posy_v[pl.ds(t * L, L)]
                p0 = p0_v[pl.ds(t * L, L)]
                lane = lax.iota(jnp.int32, L)
                for k in range(TAPS):
                    j, i = k // 3, k % 3
                    y = py + (j - 1)
                    x = px + (i - 1)
                    oob = (y < 0) | (y >= rW) | (x < 0) | (x >= rW)
                    xc = jnp.clip(x, 0, rW - 1)
                    dr = jnp.clip(y, 0, rW - 1) - 2 * p0
                    lrow = (dr >> 1) * CN + t * L + lane
                    lcol = (dr & 1) * rW + xc
                    cid = plsc.load_gather(rows_v, [lrow, lcol])
                    yidx_v[k, pl.ds(t * L, L)] = jnp.where(oob, pad_v, cid)

            copies = []
            for k in range(TAPS):
                copies.append(pltpu.async_copy(
                    y_hbm.at[k].at[yidx_v.at[k]],
                    ybuf_v.at[pl.ds(k * CN, CN), :], sem))
            for cp in copies:
                cp.wait()

            def acc_body(n, carry2):
                for c in range(8):
                    acc = bias_v[pl.ds(c * L, L)]
                    for k in range(TAPS):
                        acc = acc + ybuf_v[k * CN + n, pl.ds(c * L, L)]
                    outbuf_v[n, pl.ds(c * L, L)] = acc
                return carry2
            lax.fori_loop(0, CN, acc_body, 0)

            pltpu.sync_copy(outbuf_v, out_hbm.at[pl.ds(base, CN), :])
            return carry

        lax.fori_loop(0, chunks, chunk_body, 0)

    return sc_kernel


def kernel(in_core_feats, aux_feats, id_map, roi_ids, pos_ids, weight, bias):
    N, C = in_core_feats.shape
    A = aux_feats.shape[0]
    O = weight.shape[0]
    R, rH, rW = id_map.shape
    V = N + A + 1
    pad_v = N + A

    BM = 512
    Vp = (V + BM - 1) // BM * BM
    all_feats = jnp.concatenate(
        [in_core_feats, aux_feats,
         jnp.zeros((Vp - N - A, C), jnp.float32)], axis=0)
    wbig = weight.reshape(O, TAPS, C).transpose(2, 1, 0).reshape(C, TAPS * O)
    y = _compute_y(all_feats, wbig)

    Np = (N + NW * CN - 1) // (NW * CN) * (NW * CN)
    pad_n = Np - N
    roi_p = jnp.pad(roi_ids, (0, pad_n))
    posx_p = jnp.pad(pos_ids[:, 0], (0, pad_n))
    posy_p = jnp.pad(pos_ids[:, 1], (0, pad_n))
    idmap_pairs = id_map.reshape(R * rH // 2, 2 * rW)

    sc = _make_sc_gather(Np, rW, pad_v)
    out = sc(idmap_pairs, roi_p, posx_p, posy_p, y, bias)
    return out[:N]


# speedup vs baseline: 4.0846x; 1.1250x over previous
import functools

import jax
import jax.numpy as jnp
from jax import lax
from jax.experimental import pallas as pl
from jax.experimental.pallas import tpu as pltpu
from jax.experimental.pallas import tpu_sc as plsc

L = 16
NC, NS = 2, 16
NW = NC * NS
CN = 32
TAPS = 9


def _matmul_body(x_ref, w_ref, o_ref):
    res = jnp.dot(x_ref[...], w_ref[...], preferred_element_type=jnp.float32)
    for k in range(TAPS):
        o_ref[k, ...] = res[:, k * 128:(k + 1) * 128]


def _compute_y(all_feats, wbig):
    Vp, C = all_feats.shape
    BM = 512
    return pl.pallas_call(
        _matmul_body,
        grid=(Vp // BM,),
        in_specs=[
            pl.BlockSpec((BM, C), lambda i: (i, 0)),
            pl.BlockSpec((C, TAPS * C), lambda i: (0, 0)),
        ],
        out_specs=pl.BlockSpec((TAPS, BM, C), lambda i: (0, i, 0)),
        out_shape=jax.ShapeDtypeStruct((TAPS, Vp, C), jnp.float32),
    )(all_feats, wbig)


def _make_sc_gather(Np, rW, pad_v):
    per_w = Np // NW
    chunks = per_w // CN
    assert chunks % 2 == 0
    mesh = plsc.VectorSubcoreMesh(core_axis_name="c", subcore_axis_name="s")

    buf_t = lambda *s: [pltpu.VMEM(s, jnp.int32), pltpu.VMEM(s, jnp.int32)]

    @functools.partial(
        pl.kernel,
        out_type=jax.ShapeDtypeStruct((Np, 128), jnp.float32),
        mesh=mesh,
        compiler_params=pltpu.CompilerParams(needs_layout_passes=False),
        scratch_types=[
            buf_t(3 * CN),
            buf_t(CN),
            buf_t(2 * CN),
            buf_t(2 * CN, 2 * rW),
            buf_t(TAPS, CN),
            [pltpu.VMEM((TAPS * CN, 128), jnp.float32) for _ in range(2)],
            pltpu.VMEM((128,), jnp.float32),
            [pltpu.VMEM((CN, 128), jnp.float32) for _ in range(2)],
            pltpu.SemaphoreType.DMA,
            [pltpu.SemaphoreType.DMA for _ in range(2)],
        ],
    )
    def sc_kernel(idmap_hbm, in_hbm, y_hbm, bias_hbm, out_hbm,
                  in_v, p0_v, rowidx_v, rows_v, yidx_v, ybuf_v, bias_v,
                  outbuf_v, sem_rows, sem_y):
        wid = lax.axis_index("s") * NC + lax.axis_index("c")
        w_base = wid * per_w
        pltpu.sync_copy(bias_hbm, bias_v)
        n_pairs = rW // 2

        def stage_a(g, b):
            chunk_row = wid * chunks + g
            pltpu.sync_copy(in_hbm.at[chunk_row], in_v[b])

            for t in range(CN // L):
                roi = in_v[b][pl.ds(t * L, L)]
                py = in_v[b][pl.ds(2 * CN + t * L, L)]
                ylo = jnp.clip(py - 1, 0, rW - 1)
                p0 = jnp.minimum(ylo >> 1, n_pairs - 2)
                p0_v[b][pl.ds(t * L, L)] = p0
                base_pair = roi * n_pairs + p0
                rowidx_v[b][pl.ds(t * L, L)] = base_pair
                rowidx_v[b][pl.ds(CN + t * L, L)] = base_pair + 1
            pltpu.async_copy(idmap_hbm.at[rowidx_v[b]], rows_v[b],
                             sem_rows).wait()

            for t in range(CN // L):
                px = in_v[b][pl.ds(CN + t * L, L)]
                py = in_v[b][pl.ds(2 * CN + t * L, L)]
                p0 = p0_v[b][pl.ds(t * L, L)]
                lane = lax.iota(jnp.int32, L)
                for k in range(TAPS):
                    j, i = k // 3, k % 3
                    y = py + (j - 1)
                    x = px + (i - 1)
                    oob = (y < 0) | (y >= rW) | (x < 0) | (x >= rW)
                    xc = jnp.clip(x, 0, rW - 1)
                    dr = jnp.clip(y, 0, rW - 1) - 2 * p0
                    lrow = (dr >> 1) * CN + t * L + lane
                    lcol = (dr & 1) * rW + xc
                    cid = plsc.load_gather(rows_v[b], [lrow, lcol])
                    yidx_v[b][k, pl.ds(t * L, L)] = jnp.where(oob, pad_v, cid)

            for k in range(TAPS):
                pltpu.async_copy(y_hbm.at[k].at[yidx_v[b].at[k]],
                                 ybuf_v[b].at[pl.ds(k * CN, CN), :], sem_y[b])

        def stage_b(g, b):
            base = w_base + g * CN
            for k in range(TAPS):
                pltpu.make_async_copy(
                    y_hbm.at[k].at[yidx_v[b].at[k]],
                    ybuf_v[b].at[pl.ds(k * CN, CN), :], sem_y[b]).wait()

            def acc_body(n, carry2):
                for c in range(8):
                    acc = bias_v[pl.ds(c * L, L)]
                    for k in range(TAPS):
                        acc = acc + ybuf_v[b][k * CN + n, pl.ds(c * L, L)]
                    outbuf_v[b][n, pl.ds(c * L, L)] = acc
                return carry2
            lax.fori_loop(0, CN, acc_body, 0)

            pltpu.sync_copy(outbuf_v[b], out_hbm.at[pl.ds(base, CN), :])

        stage_a(0, 0)

        def pair_body(p, carry):
            g0 = 2 * p
            stage_a(g0 + 1, 1)
            stage_b(g0, 0)

            @pl.when(g0 + 2 < chunks)
            def _():
                stage_a(g0 + 2, 0)

            stage_b(g0 + 1, 1)
            return carry

        lax.fori_loop(0, chunks // 2, pair_body, 0)

    return sc_kernel


def kernel(in_core_feats, aux_feats, id_map, roi_ids, pos_ids, weight, bias):
    N, C = in_core_feats.shape
    A = aux_feats.shape[0]
    O = weight.shape[0]
    R, rH, rW = id_map.shape
    V = N + A + 1
    pad_v = N + A

    BM = 512
    Vp = (V + BM - 1) // BM * BM
    all_feats = jnp.concatenate(
        [in_core_feats, aux_feats,
         jnp.zeros((Vp - N - A, C), jnp.float32)], axis=0)
    wbig = weight.reshape(O, TAPS, C).transpose(2, 1, 0).reshape(C, TAPS * O)
    y = _compute_y(all_feats, wbig)

    grp = NW * CN * 2
    Np = (N + grp - 1) // grp * grp
    pad_n = Np - N
    in_packed = jnp.stack([
        jnp.pad(roi_ids, (0, pad_n)).reshape(-1, CN),
        jnp.pad(pos_ids[:, 0], (0, pad_n)).reshape(-1, CN),
        jnp.pad(pos_ids[:, 1], (0, pad_n)).reshape(-1, CN),
    ], axis=1).reshape(-1, 3 * CN)
    idmap_pairs = id_map.reshape(R * rH // 2, 2 * rW)

    sc = _make_sc_gather(Np, rW, pad_v)
    out = sc(idmap_pairs, in_packed, y, bias)
    return out[:N]


# speedup vs baseline: 4.5457x; 1.1129x over previous
import functools

import jax
import jax.numpy as jnp
import numpy as np
from jax import lax
from jax.experimental import pallas as pl
from jax.experimental.pallas import tpu as pltpu
from jax.experimental.pallas import tpu_sc as plsc

L = 16
NC, NS = 2, 16
NW = NC * NS
CN = 32
TAPS = 9


def _matmul_body(x_ref, w_ref, o_ref):
    res = jnp.dot(x_ref[...], w_ref[...], preferred_element_type=jnp.float32)
    for k in range(TAPS):
        o_ref[k, ...] = res[:, k * 128:(k + 1) * 128]


def _compute_y(all_feats, wbig):
    Vp, C = all_feats.shape
    BM = 512
    return pl.pallas_call(
        _matmul_body,
        grid=(Vp // BM,),
        in_specs=[
            pl.BlockSpec((BM, C), lambda i: (i, 0)),
            pl.BlockSpec((C, TAPS * C), lambda i: (0, 0)),
        ],
        out_specs=pl.BlockSpec((TAPS, BM, C), lambda i: (0, i, 0)),
        out_shape=jax.ShapeDtypeStruct((TAPS, Vp, C), jnp.float32),
    )(all_feats, wbig)


def _make_sc_gather(Np, rW, pad_v):
    per_w = Np // NW
    chunks = per_w // CN
    assert chunks % 2 == 0
    mesh = plsc.VectorSubcoreMesh(core_axis_name="c", subcore_axis_name="s")

    buf_t = lambda *s: [pltpu.VMEM(s, jnp.int32), pltpu.VMEM(s, jnp.int32)]

    @functools.partial(
        pl.kernel,
        out_type=jax.ShapeDtypeStruct((Np, 128), jnp.float32),
        mesh=mesh,
        compiler_params=pltpu.CompilerParams(needs_layout_passes=False),
        scratch_types=[
            buf_t(3 * CN),
            buf_t(CN),
            buf_t(2 * CN),
            buf_t(2 * CN, 2 * rW),
            buf_t(TAPS, CN),
            [pltpu.VMEM((TAPS * CN, 128), jnp.float32) for _ in range(2)],
            pltpu.VMEM((128,), jnp.float32),
            [pltpu.VMEM((CN, 128), jnp.float32) for _ in range(2)],
            pltpu.SemaphoreType.DMA,
            [pltpu.SemaphoreType.DMA for _ in range(2)],
            [pltpu.SemaphoreType.DMA for _ in range(2)],
        ],
    )
    def sc_kernel(idmap_hbm, in_hbm, y_hbm, bias_hbm, out_hbm,
                  in_v, p0_v, rowidx_v, rows_v, yidx_v, ybuf_v, bias_v,
                  outbuf_v, sem_rows, sem_y, sem_out):
        wid = lax.axis_index("s") * NC + lax.axis_index("c")
        w_base = wid * per_w
        pltpu.sync_copy(bias_hbm, bias_v)
        n_pairs = rW // 2

        def stage_a(g, b):
            chunk_row = wid * chunks + g
            pltpu.sync_copy(in_hbm.at[chunk_row], in_v[b])

            for t in range(CN // L):
                roi = in_v[b][pl.ds(t * L, L)]
                py = in_v[b][pl.ds(2 * CN + t * L, L)]
                ylo = jnp.clip(py - 1, 0, rW - 1)
                p0 = jnp.minimum(ylo >> 1, n_pairs - 2)
                p0_v[b][pl.ds(t * L, L)] = p0
                base_pair = roi * n_pairs + p0
                rowidx_v[b][pl.ds(t * L, L)] = base_pair
                rowidx_v[b][pl.ds(CN + t * L, L)] = base_pair + 1
            pltpu.async_copy(idmap_hbm.at[rowidx_v[b]], rows_v[b],
                             sem_rows).wait()

            for t in range(CN // L):
                px = in_v[b][pl.ds(CN + t * L, L)]
                py = in_v[b][pl.ds(2 * CN + t * L, L)]
                p0 = p0_v[b][pl.ds(t * L, L)]
                lane = lax.iota(jnp.int32, L)
                for k in range(TAPS):
                    j, i = k // 3, k % 3
                    y = py + (j - 1)
                    x = px + (i - 1)
                    oob = (y < 0) | (y >= rW) | (x < 0) | (x >= rW)
                    xc = jnp.clip(x, 0, rW - 1)
                    dr = jnp.clip(y, 0, rW - 1) - 2 * p0
                    lrow = (dr >> 1) * CN + t * L + lane
                    lcol = (dr & 1) * rW + xc
                    cid = plsc.load_gather(rows_v[b], [lrow, lcol])
                    yidx_v[b][k, pl.ds(t * L, L)] = jnp.where(oob, pad_v, cid)

            for k in range(TAPS):
                pltpu.async_copy(y_hbm.at[k].at[yidx_v[b].at[k]],
                                 ybuf_v[b].at[pl.ds(k * CN, CN), :], sem_y[b])

        def stage_b(g, b):
            base = w_base + g * CN
            for k in range(TAPS):
                pltpu.make_async_copy(
                    y_hbm.at[k].at[yidx_v[b].at[k]],
                    ybuf_v[b].at[pl.ds(k * CN, CN), :], sem_y[b]).wait()

            @pl.when(g >= 2)
            def _():
                pltpu.make_async_copy(
                    outbuf_v[b], out_hbm.at[pl.ds(base, CN), :],
                    sem_out[b]).wait()

            bias_regs = [bias_v[pl.ds(c * L, L)] for c in range(8)]

            def acc_body(n, carry2):
                for c in range(8):
                    acc = bias_regs[c]
                    for k in range(TAPS):
                        acc = acc + ybuf_v[b][k * CN + n, pl.ds(c * L, L)]
                    outbuf_v[b][n, pl.ds(c * L, L)] = acc
                return carry2
            lax.fori_loop(0, CN, acc_body, 0)

            pltpu.async_copy(outbuf_v[b], out_hbm.at[pl.ds(base, CN), :],
                             sem_out[b])

        stage_a(0, 0)

        def pair_body(p, carry):
            g0 = 2 * p
            stage_a(g0 + 1, 1)
            stage_b(g0, 0)

            @pl.when(g0 + 2 < chunks)
            def _():
                stage_a(g0 + 2, 0)

            stage_b(g0 + 1, 1)
            return carry

        lax.fori_loop(0, chunks // 2, pair_body, 0)

        for b in range(2):
            g = chunks - 2 + b
            pltpu.make_async_copy(
                outbuf_v[b], out_hbm.at[pl.ds(w_base + g * CN, CN), :],
                sem_out[b]).wait()

    return sc_kernel


def kernel(in_core_feats, aux_feats, id_map, roi_ids, pos_ids, weight, bias):
    N, C = in_core_feats.shape
    A = aux_feats.shape[0]
    O = weight.shape[0]
    R, rH, rW = id_map.shape
    V = N + A + 1
    pad_v = N + A

    BM = 512
    Vp = (V + BM - 1) // BM * BM
    all_feats = jnp.concatenate(
        [in_core_feats, aux_feats,
         jnp.zeros((Vp - N - A, C), jnp.float32)], axis=0)
    wbig = weight.reshape(O, TAPS, C).transpose(2, 1, 0).reshape(C, TAPS * O)
    y = _compute_y(all_feats, wbig)

    grp = NW * CN * 2
    Np = (N + grp - 1) // grp * grp
    pad_n = Np - N
    in_packed = jnp.stack([
        jnp.pad(roi_ids, (0, pad_n)).reshape(-1, CN),
        jnp.pad(pos_ids[:, 0], (0, pad_n)).reshape(-1, CN),
        jnp.pad(pos_ids[:, 1], (0, pad_n)).reshape(-1, CN),
    ], axis=1).reshape(-1, 3 * CN)
    idmap_pairs = id_map.reshape(R * rH // 2, 2 * rW)

    sc = _make_sc_gather(Np, rW, pad_v)
    out = sc(idmap_pairs, in_packed, y, bias)
    return out[:N]


# speedup vs baseline: 4.5779x; 1.0071x over previous
import functools

import jax
import jax.numpy as jnp
import numpy as np
from jax import lax
from jax.experimental import pallas as pl
from jax.experimental.pallas import tpu as pltpu
from jax.experimental.pallas import tpu_sc as plsc

L = 16
NC, NS = 2, 16
NW = NC * NS
CN = 32
TAPS = 9


def _matmul_body(x_ref, w_ref, o_ref):
    res = jnp.dot(x_ref[...], w_ref[...], preferred_element_type=jnp.float32)
    for k in range(TAPS):
        o_ref[k, ...] = res[:, k * 128:(k + 1) * 128]


def _compute_y(all_feats, wbig):
    Vp, C = all_feats.shape
    BM = 512
    return pl.pallas_call(
        _matmul_body,
        grid=(Vp // BM,),
        in_specs=[
            pl.BlockSpec((BM, C), lambda i: (i, 0)),
            pl.BlockSpec((C, TAPS * C), lambda i: (0, 0)),
        ],
        out_specs=pl.BlockSpec((TAPS, BM, C), lambda i: (0, i, 0)),
        out_shape=jax.ShapeDtypeStruct((TAPS, Vp, C), jnp.float32),
    )(all_feats, wbig)


def _make_sc_gather(Np, rW, pad_v):
    per_w = Np // NW
    chunks = per_w // CN
    assert chunks % 2 == 0
    mesh = plsc.VectorSubcoreMesh(core_axis_name="c", subcore_axis_name="s")

    buf_t = lambda *s: [pltpu.VMEM(s, jnp.int32), pltpu.VMEM(s, jnp.int32)]

    @functools.partial(
        pl.kernel,
        out_type=jax.ShapeDtypeStruct((Np, 128), jnp.float32),
        mesh=mesh,
        compiler_params=pltpu.CompilerParams(needs_layout_passes=False),
        scratch_types=[
            buf_t(3 * CN),
            buf_t(CN),
            buf_t(2 * CN),
            buf_t(2 * CN, 2 * rW),
            buf_t(TAPS, CN),
            [pltpu.VMEM((TAPS * CN, 128), jnp.float32) for _ in range(2)],
            pltpu.VMEM((128,), jnp.float32),
            [pltpu.VMEM((CN, 128), jnp.float32) for _ in range(2)],
            [pltpu.SemaphoreType.DMA for _ in range(2)],
            [pltpu.SemaphoreType.DMA for _ in range(2)],
            [pltpu.SemaphoreType.DMA for _ in range(2)],
        ],
    )
    def sc_kernel(idmap_hbm, in_hbm, y_hbm, bias_hbm, out_hbm,
                  in_v, p0_v, rowidx_v, rows_v, yidx_v, ybuf_v, bias_v,
                  outbuf_v, sem_rows, sem_y, sem_out):
        wid = lax.axis_index("s") * NC + lax.axis_index("c")
        w_base = wid * per_w
        pltpu.sync_copy(bias_hbm, bias_v)
        n_pairs = rW // 2

        def stage_a1(g, b):
            chunk_row = wid * chunks + g
            pltpu.sync_copy(in_hbm.at[chunk_row], in_v[b])

            for t in range(CN // L):
                roi = in_v[b][pl.ds(t * L, L)]
                py = in_v[b][pl.ds(2 * CN + t * L, L)]
                ylo = jnp.clip(py - 1, 0, rW - 1)
                p0 = jnp.minimum(ylo >> 1, n_pairs - 2)
                p0_v[b][pl.ds(t * L, L)] = p0
                base_pair = roi * n_pairs + p0
                rowidx_v[b][pl.ds(t * L, L)] = base_pair
                rowidx_v[b][pl.ds(CN + t * L, L)] = base_pair + 1
            pltpu.async_copy(idmap_hbm.at[rowidx_v[b]], rows_v[b],
                             sem_rows[b])

        def stage_a2(g, b):
            pltpu.make_async_copy(idmap_hbm.at[rowidx_v[b]], rows_v[b],
                                  sem_rows[b]).wait()
            for t in range(CN // L):
                px = in_v[b][pl.ds(CN + t * L, L)]
                py = in_v[b][pl.ds(2 * CN + t * L, L)]
                p0 = p0_v[b][pl.ds(t * L, L)]
                lane = lax.iota(jnp.int32, L)
                for k in range(TAPS):
                    j, i = k // 3, k % 3
                    y = py + (j - 1)
                    x = px + (i - 1)
                    oob = (y < 0) | (y >= rW) | (x < 0) | (x >= rW)
                    xc = jnp.clip(x, 0, rW - 1)
                    dr = jnp.clip(y, 0, rW - 1) - 2 * p0
                    lrow = (dr >> 1) * CN + t * L + lane
                    lcol = (dr & 1) * rW + xc
                    cid = plsc.load_gather(rows_v[b], [lrow, lcol])
                    yidx_v[b][k, pl.ds(t * L, L)] = jnp.where(oob, pad_v, cid)

            for k in range(TAPS):
                pltpu.async_copy(y_hbm.at[k].at[yidx_v[b].at[k]],
                                 ybuf_v[b].at[pl.ds(k * CN, CN), :], sem_y[b])

        def stage_b(g, b):
            base = w_base + g * CN
            for k in range(TAPS):
                pltpu.make_async_copy(
                    y_hbm.at[k].at[yidx_v[b].at[k]],
                    ybuf_v[b].at[pl.ds(k * CN, CN), :], sem_y[b]).wait()

            @pl.when(g >= 2)
            def _():
                pltpu.make_async_copy(
                    outbuf_v[b], out_hbm.at[pl.ds(base, CN), :],
                    sem_out[b]).wait()

            bias_regs = [bias_v[pl.ds(c * L, L)] for c in range(8)]

            def acc_body(n, carry2):
                for c in range(8):
                    acc = bias_regs[c]
                    for k in range(TAPS):
                        acc = acc + ybuf_v[b][k * CN + n, pl.ds(c * L, L)]
                    outbuf_v[b][n, pl.ds(c * L, L)] = acc
                return carry2
            lax.fori_loop(0, CN, acc_body, 0)

            pltpu.async_copy(outbuf_v[b], out_hbm.at[pl.ds(base, CN), :],
                             sem_out[b])

        stage_a1(0, 0)

        def pair_body(p, carry):
            g0 = 2 * p
            stage_a1(g0 + 1, 1)
            stage_a2(g0, 0)

            @pl.when(g0 > 0)
            def _():
                stage_b(g0 - 1, 1)

            @pl.when(g0 + 2 < chunks)
            def _():
                stage_a1(g0 + 2, 0)

            stage_a2(g0 + 1, 1)
            stage_b(g0, 0)
            return carry

        lax.fori_loop(0, chunks // 2, pair_body, 0)
        stage_b(chunks - 1, 1)

        for b in range(2):
            g = chunks - 2 + b
            pltpu.make_async_copy(
                outbuf_v[b], out_hbm.at[pl.ds(w_base + g * CN, CN), :],
                sem_out[b]).wait()

    return sc_kernel


def kernel(in_core_feats, aux_feats, id_map, roi_ids, pos_ids, weight, bias):
    N, C = in_core_feats.shape
    A = aux_feats.shape[0]
    O = weight.shape[0]
    R, rH, rW = id_map.shape
    V = N + A + 1
    pad_v = N + A

    BM = 512
    Vp = (V + BM - 1) // BM * BM
    all_feats = jnp.concatenate(
        [in_core_feats, aux_feats,
         jnp.zeros((Vp - N - A, C), jnp.float32)], axis=0)
    wbig = weight.reshape(O, TAPS, C).transpose(2, 1, 0).reshape(C, TAPS * O)
    y = _compute_y(all_feats, wbig)

    grp = NW * CN * 2
    Np = (N + grp - 1) // grp * grp
    pad_n = Np - N
    in_packed = jnp.stack([
        jnp.pad(roi_ids, (0, pad_n)).reshape(-1, CN),
        jnp.pad(pos_ids[:, 0], (0, pad_n)).reshape(-1, CN),
        jnp.pad(pos_ids[:, 1], (0, pad_n)).reshape(-1, CN),
    ], axis=1).reshape(-1, 3 * CN)
    idmap_pairs = id_map.reshape(R * rH // 2, 2 * rW)

    sc = _make_sc_gather(Np, rW, pad_v)
    out = sc(idmap_pairs, in_packed, y, bias)
    return out[:N]


# speedup vs baseline: 4.5849x; 1.0015x over previous
import functools

import jax
import jax.numpy as jnp
import numpy as np
from jax import lax
from jax.experimental import pallas as pl
from jax.experimental.pallas import tpu as pltpu
from jax.experimental.pallas import tpu_sc as plsc

L = 16
NC, NS = 2, 16
NW = NC * NS
CN = 32
TAPS = 9


def _matmul_body(x_ref, w_ref, o_ref):
    res = jnp.dot(x_ref[...], w_ref[...], preferred_element_type=jnp.float32)
    for k in range(TAPS):
        o_ref[k, ...] = res[:, k * 128:(k + 1) * 128]


def _compute_y(all_feats, wbig):
    Vp, C = all_feats.shape
    BM = 512
    return pl.pallas_call(
        _matmul_body,
        grid=(Vp // BM,),
        in_specs=[
            pl.BlockSpec((BM, C), lambda i: (i, 0)),
            pl.BlockSpec((C, TAPS * C), lambda i: (0, 0)),
        ],
        out_specs=pl.BlockSpec((TAPS, BM, C), lambda i: (0, i, 0)),
        out_shape=jax.ShapeDtypeStruct((TAPS, Vp, C), jnp.float32),
    )(all_feats, wbig)


def _make_sc_gather(Np, rW, pad_v, Vp):
    per_w = Np // NW
    chunks = per_w // CN
    assert chunks % 2 == 0
    mesh = plsc.VectorSubcoreMesh(core_axis_name="c", subcore_axis_name="s")

    buf_t = lambda *s: [pltpu.VMEM(s, jnp.int32), pltpu.VMEM(s, jnp.int32)]

    @functools.partial(
        pl.kernel,
        out_type=jax.ShapeDtypeStruct((Np, 128), jnp.float32),
        mesh=mesh,
        compiler_params=pltpu.CompilerParams(needs_layout_passes=False),
        scratch_types=[
            buf_t(3 * CN),
            buf_t(CN),
            buf_t(2 * CN),
            buf_t(2 * CN, 2 * rW),
            buf_t(3, 3 * CN),
            [pltpu.VMEM((TAPS * CN, 128), jnp.float32) for _ in range(2)],
            pltpu.VMEM((128,), jnp.float32),
            [pltpu.VMEM((CN, 128), jnp.float32) for _ in range(2)],
            [pltpu.SemaphoreType.DMA for _ in range(2)],
            [pltpu.SemaphoreType.DMA for _ in range(2)],
            [pltpu.SemaphoreType.DMA for _ in range(2)],
        ],
    )
    def sc_kernel(idmap_hbm, in_hbm, y_hbm, bias_hbm, out_hbm,
                  in_v, p0_v, rowidx_v, rows_v, yidx_v, ybuf_v, bias_v,
                  outbuf_v, sem_rows, sem_y, sem_out):
        wid = lax.axis_index("s") * NC + lax.axis_index("c")
        w_base = wid * per_w
        pltpu.sync_copy(bias_hbm, bias_v)
        n_pairs = rW // 2

        def stage_a1(g, b):
            chunk_row = wid * chunks + g
            pltpu.sync_copy(in_hbm.at[chunk_row], in_v[b])

            for t in range(CN // L):
                roi = in_v[b][pl.ds(t * L, L)]
                py = in_v[b][pl.ds(2 * CN + t * L, L)]
                ylo = jnp.clip(py - 1, 0, rW - 1)
                p0 = jnp.minimum(ylo >> 1, n_pairs - 2)
                p0_v[b][pl.ds(t * L, L)] = p0
                base_pair = roi * n_pairs + p0
                rowidx_v[b][pl.ds(t * L, L)] = base_pair
                rowidx_v[b][pl.ds(CN + t * L, L)] = base_pair + 1
            pltpu.async_copy(idmap_hbm.at[rowidx_v[b]], rows_v[b],
                             sem_rows[b])

        def stage_a2(g, b):
            pltpu.make_async_copy(idmap_hbm.at[rowidx_v[b]], rows_v[b],
                                  sem_rows[b]).wait()
            for t in range(CN // L):
                px = in_v[b][pl.ds(CN + t * L, L)]
                py = in_v[b][pl.ds(2 * CN + t * L, L)]
                p0 = p0_v[b][pl.ds(t * L, L)]
                lane = lax.iota(jnp.int32, L)
                for k in range(TAPS):
                    j, i = k // 3, k % 3
                    y = py + (j - 1)
                    x = px + (i - 1)
                    oob = (y < 0) | (y >= rW) | (x < 0) | (x >= rW)
                    xc = jnp.clip(x, 0, rW - 1)
                    dr = jnp.clip(y, 0, rW - 1) - 2 * p0
                    lrow = (dr >> 1) * CN + t * L + lane
                    lcol = (dr & 1) * rW + xc
                    cid = plsc.load_gather(rows_v[b], [lrow, lcol])
                    p = k * CN + t * L
                    yidx_v[b][p // 96, pl.ds(p % 96, L)] = (
                        jnp.where(oob, pad_v, cid) + k * Vp)

            for grp in range(3):
                pltpu.async_copy(y_hbm.at[yidx_v[b].at[grp]],
                                 ybuf_v[b].at[pl.ds(grp * 96, 96), :],
                                 sem_y[b])

        def stage_b(g, b):
            base = w_base + g * CN
            for grp in range(3):
                pltpu.make_async_copy(
                    y_hbm.at[yidx_v[b].at[grp]],
                    ybuf_v[b].at[pl.ds(grp * 96, 96), :], sem_y[b]).wait()

            @pl.when(g >= 2)
            def _():
                pltpu.make_async_copy(
                    outbuf_v[b], out_hbm.at[pl.ds(base, CN), :],
                    sem_out[b]).wait()

            bias_regs = [bias_v[pl.ds(c * L, L)] for c in range(8)]

            def acc_body(n, carry2):
                for c in range(8):
                    acc = bias_regs[c]
                    for k in range(TAPS):
                        acc = acc + ybuf_v[b][k * CN + n, pl.ds(c * L, L)]
                    outbuf_v[b][n, pl.ds(c * L, L)] = acc
                return carry2
            lax.fori_loop(0, CN, acc_body, 0)

            pltpu.async_copy(outbuf_v[b], out_hbm.at[pl.ds(base, CN), :],
                             sem_out[b])

        stage_a1(0, 0)

        def pair_body(p, carry):
            g0 = 2 * p
            stage_a1(g0 + 1, 1)
            stage_a2(g0, 0)

            @pl.when(g0 > 0)
            def _():
                stage_b(g0 - 1, 1)

            @pl.when(g0 + 2 < chunks)
            def _():
                stage_a1(g0 + 2, 0)

            stage_a2(g0 + 1, 1)
            stage_b(g0, 0)
            return carry

        lax.fori_loop(0, chunks // 2, pair_body, 0)
        stage_b(chunks - 1, 1)

        for b in range(2):
            g = chunks - 2 + b
            pltpu.make_async_copy(
                outbuf_v[b], out_hbm.at[pl.ds(w_base + g * CN, CN), :],
                sem_out[b]).wait()

    return sc_kernel


def kernel(in_core_feats, aux_feats, id_map, roi_ids, pos_ids, weight, bias):
    N, C = in_core_feats.shape
    A = aux_feats.shape[0]
    O = weight.shape[0]
    R, rH, rW = id_map.shape
    V = N + A + 1
    pad_v = N + A

    BM = 512
    Vp = (V + BM - 1) // BM * BM
    all_feats = jnp.concatenate(
        [in_core_feats, aux_feats,
         jnp.zeros((Vp - N - A, C), jnp.float32)], axis=0)
    wbig = weight.reshape(O, TAPS, C).transpose(2, 1, 0).reshape(C, TAPS * O)
    y = _compute_y(all_feats, wbig)

    grp = NW * CN * 2
    Np = (N + grp - 1) // grp * grp
    pad_n = Np - N
    in_packed = jnp.stack([
        jnp.pad(roi_ids, (0, pad_n)).reshape(-1, CN),
        jnp.pad(pos_ids[:, 0], (0, pad_n)).reshape(-1, CN),
        jnp.pad(pos_ids[:, 1], (0, pad_n)).reshape(-1, CN),
    ], axis=1).reshape(-1, 3 * CN)
    idmap_pairs = id_map.reshape(R * rH // 2, 2 * rW)

    sc = _make_sc_gather(Np, rW, pad_v, Vp)
    yflat = y.reshape(TAPS * Vp, C)
    out = sc(idmap_pairs, in_packed, yflat, bias)
    return out[:N]


# speedup vs baseline: 4.8791x; 1.0642x over previous
import functools

import jax
import jax.numpy as jnp
import numpy as np
from jax import lax
from jax.experimental import pallas as pl
from jax.experimental.pallas import tpu as pltpu
from jax.experimental.pallas import tpu_sc as plsc

L = 16
NC, NS = 2, 16
NW = NC * NS
CN = 32
TAPS = 9


def _matmul_body(x_ref, w_ref, o_ref):
    res = jnp.dot(x_ref[...], w_ref[...], preferred_element_type=jnp.float32)
    for k in range(TAPS):
        o_ref[k, ...] = res[:, k * 128:(k + 1) * 128]


def _compute_y(all_feats, wbig):
    Vp, C = all_feats.shape
    BM = 1024
    return pl.pallas_call(
        _matmul_body,
        grid=(Vp // BM,),
        in_specs=[
            pl.BlockSpec((BM, C), lambda i: (i, 0)),
            pl.BlockSpec((C, TAPS * C), lambda i: (0, 0)),
        ],
        out_specs=pl.BlockSpec((TAPS, BM, C), lambda i: (0, i, 0)),
        out_shape=jax.ShapeDtypeStruct((TAPS, Vp, C), jnp.float32),
    )(all_feats, wbig)


def _make_sc_gather(Np, rW, pad_v, Vp):
    per_w = Np // NW
    chunks = per_w // CN
    assert chunks % 2 == 0
    mesh = plsc.VectorSubcoreMesh(core_axis_name="c", subcore_axis_name="s")

    buf_t = lambda *s: [pltpu.VMEM(s, jnp.int32), pltpu.VMEM(s, jnp.int32)]

    @functools.partial(
        pl.kernel,
        out_type=jax.ShapeDtypeStruct((Np, 128), jnp.float32),
        mesh=mesh,
        compiler_params=pltpu.CompilerParams(needs_layout_passes=False),
        scratch_types=[
            buf_t(3 * CN),
            buf_t(CN),
            buf_t(2 * CN),
            buf_t(2 * CN, 2 * rW),
            buf_t(3, 3 * CN),
            [pltpu.VMEM((TAPS * CN, 128), jnp.float32) for _ in range(2)],
            pltpu.VMEM((128,), jnp.float32),
            [pltpu.VMEM((CN, 128), jnp.float32) for _ in range(2)],
            [pltpu.SemaphoreType.DMA for _ in range(2)],
            [pltpu.SemaphoreType.DMA for _ in range(2)],
            [pltpu.SemaphoreType.DMA for _ in range(2)],
        ],
    )
    def sc_kernel(idmap_hbm, in_hbm, y_hbm, bias_hbm, out_hbm,
                  in_v, p0_v, rowidx_v, rows_v, yidx_v, ybuf_v, bias_v,
                  outbuf_v, sem_rows, sem_y, sem_out):
        wid = lax.axis_index("s") * NC + lax.axis_index("c")
        w_base = wid * per_w
        pltpu.sync_copy(bias_hbm, bias_v)
        n_pairs = rW // 2

        def stage_a1(g, b):
            chunk_row = wid * chunks + g
            pltpu.sync_copy(in_hbm.at[chunk_row], in_v[b])

            for t in range(CN // L):
                roi = in_v[b][pl.ds(t * L, L)]
                py = in_v[b][pl.ds(2 * CN + t * L, L)]
                ylo = jnp.clip(py - 1, 0, rW - 1)
                p0 = jnp.minimum(ylo >> 1, n_pairs - 2)
                p0_v[b][pl.ds(t * L, L)] = p0
                base_pair = roi * n_pairs + p0
                rowidx_v[b][pl.ds(t * L, L)] = base_pair
                rowidx_v[b][pl.ds(CN + t * L, L)] = base_pair + 1
            pltpu.async_copy(idmap_hbm.at[rowidx_v[b]], rows_v[b],
                             sem_rows[b])

        def stage_a2(g, b):
            pltpu.make_async_copy(idmap_hbm.at[rowidx_v[b]], rows_v[b],
                                  sem_rows[b]).wait()
            for t in range(CN // L):
                px = in_v[b][pl.ds(CN + t * L, L)]
                py = in_v[b][pl.ds(2 * CN + t * L, L)]
                p0 = p0_v[b][pl.ds(t * L, L)]
                lane = lax.iota(jnp.int32, L)
                for k in range(TAPS):
                    j, i = k // 3, k % 3
                    y = py + (j - 1)
                    x = px + (i - 1)
                    oob = (y < 0) | (y >= rW) | (x < 0) | (x >= rW)
                    xc = jnp.clip(x, 0, rW - 1)
                    dr = jnp.clip(y, 0, rW - 1) - 2 * p0
                    lrow = (dr >> 1) * CN + t * L + lane
                    lcol = (dr & 1) * rW + xc
                    cid = plsc.load_gather(rows_v[b], [lrow, lcol])
                    p = k * CN + t * L
                    yidx_v[b][p // 96, pl.ds(p % 96, L)] = (
                        jnp.where(oob, pad_v, cid) + k * Vp)

            for grp in range(3):
                pltpu.async_copy(y_hbm.at[yidx_v[b].at[grp]],
                                 ybuf_v[b].at[pl.ds(grp * 96, 96), :],
                                 sem_y[b])

        def stage_b(g, b):
            base = w_base + g * CN
            for grp in range(3):
                pltpu.make_async_copy(
                    y_hbm.at[yidx_v[b].at[grp]],
                    ybuf_v[b].at[pl.ds(grp * 96, 96), :], sem_y[b]).wait()

            @pl.when(g >= 2)
            def _():
                pltpu.make_async_copy(
                    outbuf_v[b], out_hbm.at[pl.ds(base, CN), :],
                    sem_out[b]).wait()

            bias_regs = [bias_v[pl.ds(c * L, L)] for c in range(8)]

            def acc_body(n, carry2):
                for c in range(8):
                    acc = bias_regs[c]
                    for k in range(TAPS):
                        acc = acc + ybuf_v[b][k * CN + n, pl.ds(c * L, L)]
                    outbuf_v[b][n, pl.ds(c * L, L)] = acc
                return carry2
            lax.fori_loop(0, CN, acc_body, 0)

            pltpu.async_copy(outbuf_v[b], out_hbm.at[pl.ds(base, CN), :],
                             sem_out[b])

        stage_a1(0, 0)

        def pair_body(p, carry):
            g0 = 2 * p
            stage_a1(g0 + 1, 1)
            stage_a2(g0, 0)

            @pl.when(g0 > 0)
            def _():
                stage_b(g0 - 1, 1)

            @pl.when(g0 + 2 < chunks)
            def _():
                stage_a1(g0 + 2, 0)

            stage_a2(g0 + 1, 1)
            stage_b(g0, 0)
            return carry

        lax.fori_loop(0, chunks // 2, pair_body, 0)
        stage_b(chunks - 1, 1)

        for b in range(2):
            g = chunks - 2 + b
            pltpu.make_async_copy(
                outbuf_v[b], out_hbm.at[pl.ds(w_base + g * CN, CN), :],
                sem_out[b]).wait()

    return sc_kernel


def kernel(in_core_feats, aux_feats, id_map, roi_ids, pos_ids, weight, bias):
    N, C = in_core_feats.shape
    A = aux_feats.shape[0]
    O = weight.shape[0]
    R, rH, rW = id_map.shape
    V = N + A + 1
    pad_v = N + A

    BM = 1024
    Vp = (V + BM - 1) // BM * BM
    all_feats = jnp.concatenate(
        [in_core_feats, aux_feats,
         jnp.zeros((Vp - N - A, C), jnp.float32)], axis=0)
    wbig = weight.reshape(O, TAPS, C).transpose(2, 1, 0).reshape(C, TAPS * O)
    y = _compute_y(all_feats, wbig)

    grp = NW * CN * 2
    Np = (N + grp - 1) // grp * grp
    pad_n = Np - N
    in_packed = jnp.stack([
        jnp.pad(roi_ids, (0, pad_n)).reshape(-1, CN),
        jnp.pad(pos_ids[:, 0], (0, pad_n)).reshape(-1, CN),
        jnp.pad(pos_ids[:, 1], (0, pad_n)).reshape(-1, CN),
    ], axis=1).reshape(-1, 3 * CN)
    idmap_pairs = id_map.reshape(R * rH // 2, 2 * rW)

    sc = _make_sc_gather(Np, rW, pad_v, Vp)
    yflat = y.reshape(TAPS * Vp, C)
    out = sc(idmap_pairs, in_packed, yflat, bias)
    return out[:N]
